# baseline (speedup 1.0000x reference)
import functools

import jax
import jax.numpy as jnp
from jax import lax
from jax.experimental import pallas as pl
from jax.experimental.pallas import tpu as pltpu
from jax.experimental.pallas import tpu_sc as plsc

DIM = 1024
INTER = 704
E = 64
TOPK = 8
SI = 2 * 704
T = 2048
A = T * TOPK
ROWS_TILE = 128
MAXTILES = A // ROWS_TILE + E
CAP = MAXTILES * ROWS_TILE
TE_LEN = 264

_NC, _NS = 2, 16
_NW = _NC * _NS
_APW = A // _NW
_CH = 64
_BF = jnp.bfloat16
_F32 = jnp.float32


def _gate_route_body(xf_ref, gate_ref, dest_ref, wts_ref, te_ref, s_s):
    logits = lax.dot_general(
        xf_ref[...], gate_ref[...], (((1,), (1,)), ((), ())),
        preferred_element_type=_F32, precision=lax.Precision.HIGHEST)
    m = jnp.max(logits, axis=1, keepdims=True)
    p = jnp.exp(logits - m)
    scores = p / jnp.sum(p, axis=1, keepdims=True)
    lane = lax.broadcasted_iota(jnp.int32, (T, E), 1)

    s = scores
    hots = []
    wcols = []
    S = jnp.zeros((T, E), _F32)
    for _ in range(TOPK):
        mv = jnp.max(s, axis=1, keepdims=True)
        sel = jnp.min(jnp.where(s == mv, lane, E), axis=1, keepdims=True)
        hit = (lane == sel).astype(_F32)
        hots.append(hit)
        wcols.append(mv)
        S = S + hit
        s = jnp.where(lane == sel, -jnp.inf, s)
    wts_ref[...] = jnp.concatenate(wcols, axis=1)

    sub = lax.broadcasted_iota(jnp.int32, (ROWS_TILE, ROWS_TILE), 0)
    ln2 = lax.broadcasted_iota(jnp.int32, (ROWS_TILE, ROWS_TILE), 1)
    tril = (sub >= ln2).astype(_BF)
    run = jnp.zeros((1, E), _F32)
    G = T // ROWS_TILE
    for g in range(G):
        Sg = S[g * ROWS_TILE:(g + 1) * ROWS_TILE, :]
        Ag = lax.dot_general(tril, Sg.astype(_BF), (((1,), (0,)), ((), ())),
                             preferred_element_type=_F32)
        s_s[g * ROWS_TILE:(g + 1) * ROWS_TILE, :] = Ag - Sg + run
        run = run + Ag[ROWS_TILE - 1:ROWS_TILE, :]
    counts = run

    tiles = jnp.floor((counts + (ROWS_TILE - 1)) * (1.0 / ROWS_TILE))
    s64 = lax.broadcasted_iota(jnp.int32, (E, E), 0)
    l64 = lax.broadcasted_iota(jnp.int32, (E, E), 1)
    ut = (s64 < l64).astype(_BF)
    tile_start = lax.dot_general(tiles.astype(_BF), ut,
                                 (((1,), (0,)), ((), ())),
                                 preferred_element_type=_F32)
    tile_end = tile_start + tiles

    D = tile_start * float(ROWS_TILE) + s_s[...]
    dcols = [jnp.sum(h * D, axis=1, keepdims=True) for h in hots]
    dest_ref[...] = jnp.concatenate(dcols, axis=1).astype(jnp.int32)

    tt = lax.broadcasted_iota(jnp.int32, (2 * ROWS_TILE, E), 0)
    cmp = (tt.astype(_F32) >= jnp.broadcast_to(tile_end, (2 * ROWS_TILE, E)))
    te = jnp.sum(cmp.astype(_F32), axis=1, keepdims=True)
    te_ref[0:2 * ROWS_TILE, :] = jnp.minimum(te, float(E - 1)).astype(jnp.int32)
    nact = jnp.sum(tiles, axis=1, keepdims=True)
    te_ref[2 * ROWS_TILE:TE_LEN, :] = jnp.broadcast_to(
        nact, (TE_LEN - 2 * ROWS_TILE, 1)).astype(jnp.int32)


def _sc_dispatch_body(xf_hbm, tok_hbm, dest_hbm, xd_hbm,
                      idx_v, dst_v, rows_v, sem_g, sem_s):
    wid = lax.axis_index("s") * _NC + lax.axis_index("c")
    base = wid * _APW
    for c in range(_APW // _CH):
        off = base + c * _CH
        pltpu.sync_copy(tok_hbm.at[pl.ds(off, _CH)], idx_v)
        pltpu.sync_copy(dest_hbm.at[pl.ds(off, _CH)], dst_v)
        pltpu.async_copy(xf_hbm.at[idx_v], rows_v, sem_g).wait()
        pltpu.async_copy(rows_v, xd_hbm.at[dst_v], sem_s).wait()


def _sc_combine_body(os_hbm, dest_hbm, oc_hbm,
                     idx_v, dst_v, rows_v, sem_g, sem_s):
    wid = lax.axis_index("s") * _NC + lax.axis_index("c")
    base = wid * _APW
    for c in range(_APW // _CH):
        off = base + c * _CH
        pltpu.sync_copy(dest_hbm.at[pl.ds(off, _CH)], idx_v)
        pltpu.async_copy(os_hbm.at[idx_v], rows_v, sem_g).wait()
        pltpu.sync_copy(rows_v, oc_hbm.at[pl.ds(off, _CH)])


def _ffn_h_body(te_ref, xd_ref, w1_ref, w3_ref, g_ref):
    @pl.when(pl.program_id(0) < te_ref[2 * ROWS_TILE])
    def _():
        xb = xd_ref[...].astype(_BF)
        h1 = lax.dot_general(xb, w1_ref[...].astype(_BF),
                             (((1,), (1,)), ((), ())),
                             preferred_element_type=_F32)
        h3 = lax.dot_general(xb, w3_ref[...].astype(_BF),
                             (((1,), (1,)), ((), ())),
                             preferred_element_type=_F32)
        g_ref[...] = (h1 * lax.logistic(h1) * h3).astype(_BF)


def _ffn_o_body(te_ref, g_ref, w2_ref, o_ref):
    @pl.when(pl.program_id(0) < te_ref[2 * ROWS_TILE])
    def _():
        o_ref[...] = lax.dot_general(g_ref[...], w2_ref[...].astype(_BF),
                                     (((1,), (1,)), ((), ())),
                                     preferred_element_type=_F32)


def _shared_body(xf_ref, sw1_ref, sw2_ref, sw3_ref, z_ref):
    xb = xf_ref[...].astype(_BF)
    h1 = lax.dot_general(xb, sw1_ref[...].astype(_BF),
                         (((1,), (1,)), ((), ())),
                         preferred_element_type=_F32)
    h3 = lax.dot_general(xb, sw3_ref[...].astype(_BF),
                         (((1,), (1,)), ((), ())),
                         preferred_element_type=_F32)
    g = (h1 * lax.logistic(h1) * h3).astype(_BF)
    z_ref[...] = lax.dot_general(g, sw2_ref[...].astype(_BF),
                                 (((1,), (1,)), ((), ())),
                                 preferred_element_type=_F32)


def _combine_body(oc_ref, wts_ref, xe_ref, z_ref, o_ref):
    y = jnp.zeros(o_ref.shape, _F32)
    for k in range(TOPK):
        y = y + oc_ref[:, k, :] * wts_ref[:, k][:, None]
    o_ref[...] = xe_ref[...] + y + z_ref[...]


def kernel(x_combined, xe_current, gate_w, w1, w2, w3, sw1, sw2, sw3):
    shape = x_combined.shape
    xf = x_combined.reshape(T, DIM)
    xe = xe_current.reshape(T, DIM)

    dest, wts, te = pl.pallas_call(
        _gate_route_body,
        in_specs=[
            pl.BlockSpec((T, DIM), lambda: (0, 0)),
            pl.BlockSpec((E, DIM), lambda: (0, 0)),
        ],
        out_specs=[
            pl.BlockSpec((T, TOPK), lambda: (0, 0)),
            pl.BlockSpec((T, TOPK), lambda: (0, 0)),
            pl.BlockSpec((TE_LEN, 1), lambda: (0, 0)),
        ],
        out_shape=[
            jax.ShapeDtypeStruct((T, TOPK), jnp.int32),
            jax.ShapeDtypeStruct((T, TOPK), _F32),
            jax.ShapeDtypeStruct((TE_LEN, 1), jnp.int32),
        ],
        scratch_shapes=[pltpu.VMEM((T, E), _F32)],
        compiler_params=pltpu.CompilerParams(
            vmem_limit_bytes=96 * 1024 * 1024),
    )(xf, gate_w)

    destf = dest.reshape(A)
    tokrep = (jnp.arange(A, dtype=jnp.int32) // TOPK).astype(jnp.int32)
    te1 = te.reshape(TE_LEN)

    mesh = plsc.VectorSubcoreMesh(core_axis_name="c", subcore_axis_name="s")
    sc_scratch = [
        pltpu.VMEM((_CH,), jnp.int32),
        pltpu.VMEM((_CH,), jnp.int32),
        pltpu.VMEM((_CH, DIM), _F32),
        pltpu.SemaphoreType.DMA,
        pltpu.SemaphoreType.DMA,
    ]

    xd = pl.kernel(
        _sc_dispatch_body,
        mesh=mesh,
        out_type=jax.ShapeDtypeStruct((CAP, DIM), _F32),
        scratch_types=sc_scratch,
    )(xf, tokrep, destf)

    TT = 256
    z = pl.pallas_call(
        _shared_body,
        grid=(T // TT,),
        in_specs=[
            pl.BlockSpec((TT, DIM), lambda t: (t, 0)),
            pl.BlockSpec((SI, DIM), lambda t: (0, 0)),
            pl.BlockSpec((DIM, SI), lambda t: (0, 0)),
            pl.BlockSpec((SI, DIM), lambda t: (0, 0)),
        ],
        out_specs=pl.BlockSpec((TT, DIM), lambda t: (t, 0)),
        out_shape=jax.ShapeDtypeStruct((T, DIM), _F32),
        compiler_params=pltpu.CompilerParams(
            vmem_limit_bytes=100 * 1024 * 1024),
    )(xf, sw1, sw2, sw3)

    def _xd_map(t, te_r):
        tc = jnp.minimum(t, te_r[2 * ROWS_TILE] - 1)
        return (tc, 0)

    def _w13_map(t, te_r):
        tc = jnp.minimum(t, te_r[2 * ROWS_TILE] - 1)
        return (te_r[tc], 0)

    gbuf = pl.pallas_call(
        _ffn_h_body,
        grid_spec=pltpu.PrefetchScalarGridSpec(
            num_scalar_prefetch=1,
            grid=(MAXTILES,),
            in_specs=[
                pl.BlockSpec((ROWS_TILE, DIM), _xd_map),
                pl.BlockSpec((INTER, DIM), _w13_map),
                pl.BlockSpec((INTER, DIM), _w13_map),
            ],
            out_specs=pl.BlockSpec((ROWS_TILE, INTER), _xd_map),
        ),
        out_shape=jax.ShapeDtypeStruct((CAP, INTER), _BF),
        compiler_params=pltpu.CompilerParams(
            vmem_limit_bytes=64 * 1024 * 1024),
    )(te1, xd, w1.reshape(E * INTER, DIM), w3.reshape(E * INTER, DIM))

    os_ = pl.pallas_call(
        _ffn_o_body,
        grid_spec=pltpu.PrefetchScalarGridSpec(
            num_scalar_prefetch=1,
            grid=(MAXTILES,),
            in_specs=[
                pl.BlockSpec((ROWS_TILE, INTER), _xd_map),
                pl.BlockSpec((DIM, INTER), _w13_map),
            ],
            out_specs=pl.BlockSpec((ROWS_TILE, DIM), _xd_map),
        ),
        out_shape=jax.ShapeDtypeStruct((CAP, DIM), _F32),
        compiler_params=pltpu.CompilerParams(
            vmem_limit_bytes=64 * 1024 * 1024),
    )(te1, gbuf, w2.reshape(E * DIM, INTER))

    oc = pl.kernel(
        _sc_combine_body,
        mesh=mesh,
        out_type=jax.ShapeDtypeStruct((A, DIM), _F32),
        scratch_types=sc_scratch,
    )(os_, destf)

    oc3 = oc.reshape(T, TOPK, DIM)
    out = pl.pallas_call(
        _combine_body,
        grid=(T // TT,),
        in_specs=[
            pl.BlockSpec((TT, TOPK, DIM), lambda t: (t, 0, 0)),
            pl.BlockSpec((TT, TOPK), lambda t: (t, 0)),
            pl.BlockSpec((TT, DIM), lambda t: (t, 0)),
            pl.BlockSpec((TT, DIM), lambda t: (t, 0)),
        ],
        out_specs=pl.BlockSpec((TT, DIM), lambda t: (t, 0)),
        out_shape=jax.ShapeDtypeStruct((T, DIM), _F32),
        compiler_params=pltpu.CompilerParams(
            vmem_limit_bytes=100 * 1024 * 1024),
    )(oc3, wts, xe, z)

    return out.reshape(shape)


# speedup vs baseline: 1.3251x; 1.3251x over previous
import functools

import jax
import jax.numpy as jnp
from jax import lax
from jax.experimental import pallas as pl
from jax.experimental.pallas import tpu as pltpu
from jax.experimental.pallas import tpu_sc as plsc

DIM = 1024
INTER = 704
E = 64
TOPK = 8
SI = 2 * 704
T = 2048
A = T * TOPK
ROWS_TILE = 256
MAXTILES = A // ROWS_TILE + E
CAP = MAXTILES * ROWS_TILE
TE_LEN = MAXTILES + 8

_NC, _NS = 2, 16
_NW = _NC * _NS
_APW = A // _NW
_CH = 64
_BF = jnp.bfloat16
_F32 = jnp.float32


def _gate_route_body(xf_ref, gate_ref, dest_ref, wts_ref, te_ref, s_s):
    logits = lax.dot_general(
        xf_ref[...], gate_ref[...], (((1,), (1,)), ((), ())),
        preferred_element_type=_F32, precision=lax.Precision.HIGHEST)
    m = jnp.max(logits, axis=1, keepdims=True)
    p = jnp.exp(logits - m)
    scores = p / jnp.sum(p, axis=1, keepdims=True)
    lane = lax.broadcasted_iota(jnp.int32, (T, E), 1)

    s = scores
    hots = []
    wcols = []
    S = jnp.zeros((T, E), _F32)
    for _ in range(TOPK):
        mv = jnp.max(s, axis=1, keepdims=True)
        sel = jnp.min(jnp.where(s == mv, lane, E), axis=1, keepdims=True)
        hit = (lane == sel).astype(_F32)
        hots.append(hit)
        wcols.append(mv)
        S = S + hit
        s = jnp.where(lane == sel, -jnp.inf, s)
    wts_ref[...] = jnp.concatenate(wcols, axis=1)

    GB = 128
    sub = lax.broadcasted_iota(jnp.int32, (GB, GB), 0)
    ln2 = lax.broadcasted_iota(jnp.int32, (GB, GB), 1)
    tril = (sub >= ln2).astype(_BF)
    run = jnp.zeros((1, E), _F32)
    G = T // GB
    for g in range(G):
        Sg = S[g * GB:(g + 1) * GB, :]
        Ag = lax.dot_general(tril, Sg.astype(_BF), (((1,), (0,)), ((), ())),
                             preferred_element_type=_F32)
        s_s[g * GB:(g + 1) * GB, :] = Ag - Sg + run
        run = run + Ag[GB - 1:GB, :]
    counts = run

    tiles = jnp.floor((counts + (ROWS_TILE - 1)) * (1.0 / ROWS_TILE))
    s64 = lax.broadcasted_iota(jnp.int32, (E, E), 0)
    l64 = lax.broadcasted_iota(jnp.int32, (E, E), 1)
    ut = (s64 < l64).astype(_BF)
    tile_start = lax.dot_general(tiles.astype(_BF), ut,
                                 (((1,), (0,)), ((), ())),
                                 preferred_element_type=_F32)
    tile_end = tile_start + tiles

    D = tile_start * float(ROWS_TILE) + s_s[...]
    dcols = [jnp.sum(h * D, axis=1, keepdims=True) for h in hots]
    dest_ref[...] = jnp.concatenate(dcols, axis=1).astype(jnp.int32)

    tt = lax.broadcasted_iota(jnp.int32, (MAXTILES, E), 0)
    cmp = (tt.astype(_F32) >= jnp.broadcast_to(tile_end, (MAXTILES, E)))
    te = jnp.sum(cmp.astype(_F32), axis=1, keepdims=True)
    te_ref[0:MAXTILES, :] = jnp.minimum(te, float(E - 1)).astype(jnp.int32)
    nact = jnp.sum(tiles, axis=1, keepdims=True)
    te_ref[MAXTILES:TE_LEN, :] = jnp.broadcast_to(
        nact, (TE_LEN - MAXTILES, 1)).astype(jnp.int32)


def _sc_dispatch_body(xf_hbm, tok_hbm, dest_hbm, xd_hbm,
                      idx_v, dst_v, rows_v, sem_g, sem_s):
    wid = lax.axis_index("s") * _NC + lax.axis_index("c")
    base = wid * _APW
    for c in range(_APW // _CH):
        off = base + c * _CH
        pltpu.sync_copy(tok_hbm.at[pl.ds(off, _CH)], idx_v)
        pltpu.sync_copy(dest_hbm.at[pl.ds(off, _CH)], dst_v)
        pltpu.async_copy(xf_hbm.at[idx_v], rows_v, sem_g).wait()
        pltpu.async_copy(rows_v, xd_hbm.at[dst_v], sem_s).wait()


def _sc_combine_body(os_hbm, dest_hbm, oc_hbm,
                     idx_v, dst_v, rows_v, sem_g, sem_s):
    wid = lax.axis_index("s") * _NC + lax.axis_index("c")
    base = wid * _APW
    for c in range(_APW // _CH):
        off = base + c * _CH
        pltpu.sync_copy(dest_hbm.at[pl.ds(off, _CH)], idx_v)
        pltpu.async_copy(os_hbm.at[idx_v], rows_v, sem_g).wait()
        pltpu.sync_copy(rows_v, oc_hbm.at[pl.ds(off, _CH)])


def _ffn_body(te_ref, xd_ref, w1_ref, w3_ref, w2_ref, o_ref):
    @pl.when(pl.program_id(0) < te_ref[MAXTILES])
    def _():
        xb = xd_ref[...].astype(_BF)
        h1 = lax.dot_general(xb, w1_ref[...].astype(_BF),
                             (((1,), (1,)), ((), ())),
                             preferred_element_type=_F32)
        h3 = lax.dot_general(xb, w3_ref[...].astype(_BF),
                             (((1,), (1,)), ((), ())),
                             preferred_element_type=_F32)
        g = (h1 * lax.logistic(h1) * h3).astype(_BF)
        o_ref[...] = lax.dot_general(g, w2_ref[...].astype(_BF),
                                     (((1,), (1,)), ((), ())),
                                     preferred_element_type=_F32)


def _shared_body(xf_ref, sw1_ref, sw2_ref, sw3_ref, z_ref):
    xb = xf_ref[...].astype(_BF)
    h1 = lax.dot_general(xb, sw1_ref[...].astype(_BF),
                         (((1,), (1,)), ((), ())),
                         preferred_element_type=_F32)
    h3 = lax.dot_general(xb, sw3_ref[...].astype(_BF),
                         (((1,), (1,)), ((), ())),
                         preferred_element_type=_F32)
    g = (h1 * lax.logistic(h1) * h3).astype(_BF)
    z_ref[...] = lax.dot_general(g, sw2_ref[...].astype(_BF),
                                 (((1,), (1,)), ((), ())),
                                 preferred_element_type=_F32)


def _combine_body(oc_ref, wts_ref, xe_ref, z_ref, o_ref):
    y = jnp.zeros(o_ref.shape, _F32)
    for k in range(TOPK):
        y = y + oc_ref[:, k, :] * wts_ref[:, k][:, None]
    o_ref[...] = xe_ref[...] + y + z_ref[...]


def kernel(x_combined, xe_current, gate_w, w1, w2, w3, sw1, sw2, sw3):
    shape = x_combined.shape
    xf = x_combined.reshape(T, DIM)
    xe = xe_current.reshape(T, DIM)

    dest, wts, te = pl.pallas_call(
        _gate_route_body,
        in_specs=[
            pl.BlockSpec((T, DIM), lambda: (0, 0)),
            pl.BlockSpec((E, DIM), lambda: (0, 0)),
        ],
        out_specs=[
            pl.BlockSpec((T, TOPK), lambda: (0, 0)),
            pl.BlockSpec((T, TOPK), lambda: (0, 0)),
            pl.BlockSpec((TE_LEN, 1), lambda: (0, 0)),
        ],
        out_shape=[
            jax.ShapeDtypeStruct((T, TOPK), jnp.int32),
            jax.ShapeDtypeStruct((T, TOPK), _F32),
            jax.ShapeDtypeStruct((TE_LEN, 1), jnp.int32),
        ],
        scratch_shapes=[pltpu.VMEM((T, E), _F32)],
        compiler_params=pltpu.CompilerParams(
            vmem_limit_bytes=96 * 1024 * 1024),
    )(xf, gate_w)

    destf = dest.reshape(A)
    tokrep = (jnp.arange(A, dtype=jnp.int32) // TOPK).astype(jnp.int32)
    te1 = te.reshape(TE_LEN)

    mesh = plsc.VectorSubcoreMesh(core_axis_name="c", subcore_axis_name="s")
    sc_scratch = [
        pltpu.VMEM((_CH,), jnp.int32),
        pltpu.VMEM((_CH,), jnp.int32),
        pltpu.VMEM((_CH, DIM), _F32),
        pltpu.SemaphoreType.DMA,
        pltpu.SemaphoreType.DMA,
    ]

    xd = pl.kernel(
        _sc_dispatch_body,
        mesh=mesh,
        out_type=jax.ShapeDtypeStruct((CAP, DIM), _F32),
        scratch_types=sc_scratch,
    )(xf, tokrep, destf)

    TT = 256
    z = pl.pallas_call(
        _shared_body,
        grid=(T // TT,),
        in_specs=[
            pl.BlockSpec((TT, DIM), lambda t: (t, 0)),
            pl.BlockSpec((SI, DIM), lambda t: (0, 0)),
            pl.BlockSpec((DIM, SI), lambda t: (0, 0)),
            pl.BlockSpec((SI, DIM), lambda t: (0, 0)),
        ],
        out_specs=pl.BlockSpec((TT, DIM), lambda t: (t, 0)),
        out_shape=jax.ShapeDtypeStruct((T, DIM), _F32),
        compiler_params=pltpu.CompilerParams(
            vmem_limit_bytes=100 * 1024 * 1024),
    )(xf, sw1, sw2, sw3)

    def _xd_map(t, te_r):
        tc = jnp.minimum(t, te_r[MAXTILES] - 1)
        return (tc, 0)

    def _w13_map(t, te_r):
        tc = jnp.minimum(t, te_r[MAXTILES] - 1)
        return (te_r[tc], 0)

    os_ = pl.pallas_call(
        _ffn_body,
        grid_spec=pltpu.PrefetchScalarGridSpec(
            num_scalar_prefetch=1,
            grid=(MAXTILES,),
            in_specs=[
                pl.BlockSpec((ROWS_TILE, DIM), _xd_map),
                pl.BlockSpec((INTER, DIM), _w13_map),
                pl.BlockSpec((INTER, DIM), _w13_map),
                pl.BlockSpec((DIM, INTER), _w13_map),
            ],
            out_specs=pl.BlockSpec((ROWS_TILE, DIM), _xd_map),
        ),
        out_shape=jax.ShapeDtypeStruct((CAP, DIM), _F32),
        compiler_params=pltpu.CompilerParams(
            vmem_limit_bytes=64 * 1024 * 1024),
    )(te1, xd, w1.reshape(E * INTER, DIM), w3.reshape(E * INTER, DIM),
      w2.reshape(E * DIM, INTER))

    oc = pl.kernel(
        _sc_combine_body,
        mesh=mesh,
        out_type=jax.ShapeDtypeStruct((A, DIM), _F32),
        scratch_types=sc_scratch,
    )(os_, destf)

    oc3 = oc.reshape(T, TOPK, DIM)
    out = pl.pallas_call(
        _combine_body,
        grid=(T // TT,),
        in_specs=[
            pl.BlockSpec((TT, TOPK, DIM), lambda t: (t, 0, 0)),
            pl.BlockSpec((TT, TOPK), lambda t: (t, 0)),
            pl.BlockSpec((TT, DIM), lambda t: (t, 0)),
            pl.BlockSpec((TT, DIM), lambda t: (t, 0)),
        ],
        out_specs=pl.BlockSpec((TT, DIM), lambda t: (t, 0)),
        out_shape=jax.ShapeDtypeStruct((T, DIM), _F32),
        compiler_params=pltpu.CompilerParams(
            vmem_limit_bytes=100 * 1024 * 1024),
    )(oc3, wts, xe, z)

    return out.reshape(shape)
